# Initial kernel scaffold; baseline (speedup 1.0000x reference)
import jax
import jax.numpy as jnp
from jax.experimental import pallas as pl


def kernel(samples, tuple_mapping, memory):
    raise NotImplementedError("write your pallas kernel here")


# speedup vs baseline: 2.2657x; 2.2657x over previous
import functools

import jax
import jax.numpy as jnp
from jax import lax
from jax.experimental import pallas as pl
from jax.experimental.pallas import tpu as pltpu
from jax.experimental.pallas import tpu_sc as plsc

LANES = 16
NWORK = 32


def _addr_body(w_ref, x_ref, o_ref):
    acc = lax.dot_general(
        w_ref[...], x_ref[...], (((1,), (0,)), ((), ())),
        preferred_element_type=jnp.float32)
    o_ref[...] = acc.astype(jnp.int32)


def _reduce_body(p_ref, o_ref):
    o_ref[...] = jnp.sum(p_ref[...], axis=0)


def _make_sc_gather(n_rows, n_cls, n_addr, batch):
    n_neu = n_rows // n_cls
    grp = batch // LANES
    mesh = plsc.VectorSubcoreMesh(core_axis_name="c", subcore_axis_name="s")

    @functools.partial(
        pl.kernel,
        out_type=jax.ShapeDtypeStruct((NWORK, n_cls, batch), jnp.float32),
        mesh=mesh,
        compiler_params=pltpu.CompilerParams(needs_layout_passes=False),
        scratch_types=[
            pltpu.VMEM((n_addr,), jnp.float32),
            pltpu.VMEM((batch,), jnp.int32),
            pltpu.VMEM((n_cls, batch), jnp.float32),
        ],
    )
    def sc_gather(mem_hbm, addr_hbm, out_hbm, row_v, idx_v, acc_v):
        wid = lax.axis_index("s") * 2 + lax.axis_index("c")
        zero16 = jnp.zeros((LANES,), jnp.float32)

        for c in range(n_cls):
            def zbody(g, _, c=c):
                acc_v[c, pl.ds(g * LANES, LANES)] = zero16
                return 0
            lax.fori_loop(0, grp, zbody, 0)

        for c in range(n_cls):
            n_mine = (n_neu - wid + NWORK - 1) // NWORK

            def nbody(k, _, c=c):
                r = c * n_neu + wid + k * NWORK
                pltpu.sync_copy(addr_hbm.at[r], idx_v)
                pltpu.sync_copy(mem_hbm.at[r], row_v)

                def gbody(g, _):
                    sl = pl.ds(g * LANES, LANES)
                    idx = idx_v[sl]
                    vals = plsc.load_gather(row_v, [idx])
                    acc_v[c, sl] = acc_v[c, sl] + vals
                    return 0

                lax.fori_loop(0, grp, gbody, 0)
                return 0

            lax.fori_loop(0, n_mine, nbody, 0)

        pltpu.sync_copy(acc_v, out_hbm.at[wid])

    return sc_gather


def kernel(samples, tuple_mapping, memory):
    n_cls, n_neu, n_addr = memory.shape
    batch, entry = samples.shape
    total = tuple_mapping.shape[1]
    tup = total // n_neu
    n_rows = n_cls * n_neu

    cs = jnp.broadcast_to(jnp.arange(n_cls)[:, None], (n_cls, total))
    ns = jnp.broadcast_to(jnp.arange(total) // tup, (n_cls, total))
    ws = jnp.broadcast_to(
        (2.0 ** (tup - 1 - (jnp.arange(total) % tup))).astype(jnp.bfloat16),
        (n_cls, total))
    w = jnp.zeros((n_cls, n_neu, total), jnp.bfloat16)
    w = w.at[cs, ns, tuple_mapping].set(ws).reshape(n_rows, total)

    pad_t = jnp.concatenate(
        [samples.T.astype(jnp.bfloat16),
         jnp.zeros((total - entry, batch), jnp.bfloat16)], axis=0)

    nb = 8
    bn = batch // nb
    addr_t = pl.pallas_call(
        _addr_body,
        grid=(nb,),
        in_specs=[
            pl.BlockSpec((n_rows, total), lambda i: (0, 0)),
            pl.BlockSpec((total, bn), lambda i: (0, i)),
        ],
        out_specs=pl.BlockSpec((n_rows, bn), lambda i: (0, i)),
        out_shape=jax.ShapeDtypeStruct((n_rows, batch), jnp.int32),
    )(w, pad_t)

    partials = _make_sc_gather(n_rows, n_cls, n_addr, batch)(
        memory.reshape(n_rows, n_addr), addr_t)

    resp = pl.pallas_call(
        _reduce_body,
        out_shape=jax.ShapeDtypeStruct((n_cls, batch), jnp.float32),
    )(partials)
    return resp.T


# speedup vs baseline: 4.3916x; 1.9383x over previous
import functools

import jax
import jax.numpy as jnp
from jax import lax
from jax.experimental import pallas as pl
from jax.experimental.pallas import tpu as pltpu
from jax.experimental.pallas import tpu_sc as plsc

LANES = 16
NWORK = 32
DIV_M = 1784
DIV_S = 18


def _make_addr_body(n_rows, entry, tup):
    def addr_body(tm_ref, x_ref, o_ref, w_ref):
        @pl.when(pl.program_id(0) == 0)
        def _():
            iota = lax.broadcasted_iota(jnp.int32, (n_rows, entry), 1)
            acc = jnp.zeros((n_rows, entry), jnp.float32)
            for t in range(tup):
                wt = jnp.float32(2.0 ** (tup - 1 - t))
                acc = jnp.where(tm_ref[:, t:t + 1] == iota, wt, acc)
            w_ref[...] = acc.astype(jnp.bfloat16)

        o_ref[...] = lax.dot_general(
            w_ref[...], x_ref[...], (((1,), (0,)), ((), ())),
            preferred_element_type=jnp.float32).astype(jnp.int32)

    return addr_body


def _reduce_body(p_ref, o_ref):
    o_ref[...] = jnp.sum(p_ref[...], axis=0)


def _make_sc_gather(n_rows, n_cls, n_addr, batch):
    grp = batch // LANES
    mesh = plsc.VectorSubcoreMesh(core_axis_name="c", subcore_axis_name="s")

    @functools.partial(
        pl.kernel,
        out_type=jax.ShapeDtypeStruct((NWORK * n_cls * batch,), jnp.float32),
        mesh=mesh,
        compiler_params=pltpu.CompilerParams(needs_layout_passes=False),
        scratch_types=[
            pltpu.VMEM((n_addr,), jnp.float32),
            pltpu.VMEM((n_addr,), jnp.float32),
            pltpu.VMEM((batch,), jnp.int32),
            pltpu.VMEM((batch,), jnp.int32),
            pltpu.VMEM((n_cls * batch,), jnp.float32),
            pltpu.SemaphoreType.DMA,
            pltpu.SemaphoreType.DMA,
            pltpu.SemaphoreType.DMA,
            pltpu.SemaphoreType.DMA,
        ],
    )
    def sc_gather(mem_hbm, addr_hbm, out_hbm, row0, row1, idx0, idx1, acc_v,
                  sa0, sm0, sa1, sm1):
        wid = lax.axis_index("s") * 2 + lax.axis_index("c")
        n_mine = (n_rows - wid + NWORK - 1) // NWORK
        bufs = ((idx0, row0, sa0, sm0), (idx1, row1, sa1, sm1))

        zero16 = jnp.zeros((LANES,), jnp.float32)

        def zbody(g, _):
            acc_v[pl.ds(g * LANES, LANES)] = zero16
            return 0

        lax.fori_loop(0, n_cls * grp, zbody, 0, unroll=8)

        def issue(m, b):
            idx_b, row_b, sa, sm = bufs[b]
            r = wid + m * NWORK
            pltpu.async_copy(addr_hbm.at[r], idx_b, sa)
            pltpu.async_copy(mem_hbm.at[r], row_b, sm)

        def wait(b):
            idx_b, row_b, sa, sm = bufs[b]
            pltpu.make_async_copy(addr_hbm.at[0], idx_b, sa).wait()
            pltpu.make_async_copy(mem_hbm.at[0], row_b, sm).wait()

        def compute(m, b):
            idx_b, row_b, _, _ = bufs[b]
            r = wid + m * NWORK
            off = ((r * DIV_M) >> DIV_S) * batch

            def gbody(g, _):
                idx = idx_b[pl.ds(g * LANES, LANES)]
                vals = plsc.load_gather(row_b, [idx])
                asl = pl.ds(off + g * LANES, LANES)
                acc_v[asl] = acc_v[asl] + vals
                return 0

            lax.fori_loop(0, grp, gbody, 0, unroll=8)

        issue(0, 0)

        def jbody(j, _):
            m0 = 2 * j
            m1 = m0 + 1
            wait(0)

            @pl.when(m1 < n_mine)
            def _():
                issue(m1, 1)

            compute(m0, 0)

            @pl.when(m1 < n_mine)
            def _():
                wait(1)

                @pl.when(m1 + 1 < n_mine)
                def _():
                    issue(m1 + 1, 0)

                compute(m1, 1)

            return 0

        lax.fori_loop(0, (n_mine + 1) // 2, jbody, 0)

        pltpu.sync_copy(acc_v, out_hbm.at[pl.ds(wid * n_cls * batch,
                                                n_cls * batch)])

    return sc_gather


def kernel(samples, tuple_mapping, memory):
    n_cls, n_neu, n_addr = memory.shape
    batch, entry = samples.shape
    total = tuple_mapping.shape[1]
    tup = total // n_neu
    n_rows = n_cls * n_neu

    tm_flat = tuple_mapping.reshape(n_rows, tup)
    x_t = samples.astype(jnp.bfloat16).T

    nb = 8
    bn = batch // nb
    addr_t = pl.pallas_call(
        _make_addr_body(n_rows, entry, tup),
        grid=(nb,),
        in_specs=[
            pl.BlockSpec((n_rows, tup), lambda i: (0, 0)),
            pl.BlockSpec((entry, bn), lambda i: (0, i)),
        ],
        out_specs=pl.BlockSpec((n_rows, bn), lambda i: (0, i)),
        out_shape=jax.ShapeDtypeStruct((n_rows, batch), jnp.int32),
        scratch_shapes=[pltpu.VMEM((n_rows, entry), jnp.bfloat16)],
    )(tm_flat, x_t)

    partials = _make_sc_gather(n_rows, n_cls, n_addr, batch)(
        memory.reshape(n_rows, n_addr), addr_t)

    resp = pl.pallas_call(
        _reduce_body,
        out_shape=jax.ShapeDtypeStruct((n_cls, batch), jnp.float32),
    )(partials.reshape(NWORK, n_cls * batch).reshape(NWORK, n_cls, batch))
    return resp.T


# speedup vs baseline: 7.3219x; 1.6673x over previous
import functools

import jax
import jax.numpy as jnp
from jax import lax
from jax.experimental import pallas as pl
from jax.experimental.pallas import tpu as pltpu
from jax.experimental.pallas import tpu_sc as plsc

LANES = 16
NWORK = 32
DIV_M = 1784
DIV_S = 18


def _make_addr_body(n_rows, entry, tup):
    def addr_body(tm_ref, x_ref, o_ref, w_ref):
        @pl.when(pl.program_id(0) == 0)
        def _():
            iota = lax.broadcasted_iota(jnp.int32, (n_rows, entry), 1)
            acc = jnp.zeros((n_rows, entry), jnp.float32)
            for t in range(tup):
                wt = jnp.float32(2.0 ** (tup - 1 - t))
                acc = jnp.where(tm_ref[:, t:t + 1] == iota, wt, acc)
            w_ref[...] = acc.astype(jnp.bfloat16)

        o_ref[...] = lax.dot_general(
            w_ref[...], x_ref[...].astype(jnp.bfloat16), (((1,), (1,)), ((), ())),
            preferred_element_type=jnp.float32).astype(jnp.int32)

    return addr_body


def _reduce_body(p_ref, o_ref):
    o_ref[...] = jnp.sum(p_ref[...], axis=0)


def _make_sc_gather(n_rows, n_cls, n_neu, n_addr, batch):
    grp = batch // LANES
    mesh = plsc.VectorSubcoreMesh(core_axis_name="c", subcore_axis_name="s")

    @functools.partial(
        pl.kernel,
        out_type=jax.ShapeDtypeStruct((NWORK, n_cls * batch), jnp.float32),
        mesh=mesh,
        compiler_params=pltpu.CompilerParams(needs_layout_passes=False),
        scratch_types=[
            pltpu.VMEM((n_addr,), jnp.float32),
            pltpu.VMEM((n_addr,), jnp.float32),
            pltpu.VMEM((batch,), jnp.int32),
            pltpu.VMEM((batch,), jnp.int32),
            pltpu.VMEM((n_cls * batch,), jnp.float32),
            pltpu.SemaphoreType.DMA,
            pltpu.SemaphoreType.DMA,
            pltpu.SemaphoreType.DMA,
            pltpu.SemaphoreType.DMA,
        ],
    )
    def sc_gather(mem_hbm, addr_hbm, out_hbm, row0, row1, idx0, idx1, acc_v,
                  sa0, sm0, sa1, sm1):
        wid = lax.axis_index("s") * 2 + lax.axis_index("c")
        n_mine = (n_rows - wid + NWORK - 1) // NWORK
        bufs = ((idx0, row0, sa0, sm0), (idx1, row1, sa1, sm1))

        zero16 = jnp.zeros((LANES,), jnp.float32)

        def zbody(g, _):
            acc_v[pl.ds(g * LANES, LANES)] = zero16
            return 0

        lax.fori_loop(0, n_cls * grp, zbody, 0, unroll=8)

        def issue(m, b):
            idx_b, row_b, sa, sm = bufs[b]
            r = wid + m * NWORK
            c = (r * DIV_M) >> DIV_S
            n = r - c * n_neu
            pltpu.async_copy(addr_hbm.at[r], idx_b, sa)
            pltpu.async_copy(mem_hbm.at[c, n], row_b, sm)

        def wait(b):
            idx_b, row_b, sa, sm = bufs[b]
            pltpu.make_async_copy(addr_hbm.at[0], idx_b, sa).wait()
            pltpu.make_async_copy(mem_hbm.at[0, 0], row_b, sm).wait()

        def compute(m, b):
            idx_b, row_b, _, _ = bufs[b]
            r = wid + m * NWORK
            off = ((r * DIV_M) >> DIV_S) * batch

            def gbody(g, _):
                idx = idx_b[pl.ds(g * LANES, LANES)]
                vals = plsc.load_gather(row_b, [idx])
                asl = pl.ds(off + g * LANES, LANES)
                acc_v[asl] = acc_v[asl] + vals
                return 0

            lax.fori_loop(0, grp, gbody, 0, unroll=8)

        issue(0, 0)

        def jbody(j, _):
            m0 = 2 * j
            m1 = m0 + 1
            wait(0)

            @pl.when(m1 < n_mine)
            def _():
                issue(m1, 1)

            compute(m0, 0)

            @pl.when(m1 < n_mine)
            def _():
                wait(1)

                @pl.when(m1 + 1 < n_mine)
                def _():
                    issue(m1 + 1, 0)

                compute(m1, 1)

            return 0

        lax.fori_loop(0, (n_mine + 1) // 2, jbody, 0)

        pltpu.sync_copy(acc_v, out_hbm.at[wid])

    return sc_gather


def kernel(samples, tuple_mapping, memory):
    n_cls, n_neu, n_addr = memory.shape
    batch, entry = samples.shape
    total = tuple_mapping.shape[1]
    tup = total // n_neu
    n_rows = n_cls * n_neu

    tm_flat = tuple_mapping.reshape(n_rows, tup)

    nb = 8
    bn = batch // nb
    addr_t = pl.pallas_call(
        _make_addr_body(n_rows, entry, tup),
        grid=(nb,),
        in_specs=[
            pl.BlockSpec((n_rows, tup), lambda i: (0, 0)),
            pl.BlockSpec((bn, entry), lambda i: (i, 0)),
        ],
        out_specs=pl.BlockSpec((n_rows, bn), lambda i: (0, i)),
        out_shape=jax.ShapeDtypeStruct((n_rows, batch), jnp.int32),
        scratch_shapes=[pltpu.VMEM((n_rows, entry), jnp.bfloat16)],
    )(tm_flat, samples)

    partials = _make_sc_gather(n_rows, n_cls, n_neu, n_addr, batch)(
        memory, addr_t)

    resp = pl.pallas_call(
        _reduce_body,
        out_shape=jax.ShapeDtypeStruct((n_cls * batch,), jnp.float32),
    )(partials)
    return resp.reshape(n_cls, batch).T
